# baseline (speedup 1.0000x reference)
import functools

import jax
import jax.numpy as jnp
from jax import lax
from jax.experimental import pallas as pl
from jax.experimental.pallas import tpu as pltpu
from jax.experimental.pallas import tpu_sc as plsc

NUM_GENRES = 20
EMBED_DIM = 240
TITLE_DIM = 768
MOVIE_FEAT_DIM = 64
BATCH = 16384

_NC = 2
_NS = 16
_NW = _NC * _NS
_B_PER_W = BATCH // _NW
_GROUPS = _B_PER_W // 16
_CHUNK_GROUPS = 8
_N_CHUNKS = _GROUPS // _CHUNK_GROUPS
_CHUNK_ELEMS = _CHUNK_GROUPS * 16 * EMBED_DIM


def _fast_rsqrt(s):
    i = plsc.bitcast(s, jnp.int32)
    i = jnp.int32(0x5F3759DF) - (i >> 1)
    y = plsc.bitcast(i, jnp.float32)
    for _ in range(3):
        y = y * (jnp.float32(1.5) - jnp.float32(0.5) * s * y * y)
    return y


def _sc_user_body(idx_hbm, tab_hbm, out_hbm, idx_v, tab_v, inv_v, rows_v, sem):
    wid = lax.axis_index("s") * _NC + lax.axis_index("c")
    base = wid * _B_PER_W

    pltpu.sync_copy(idx_hbm.at[pl.ds(base, _B_PER_W)], idx_v)
    pltpu.sync_copy(tab_hbm, tab_v)

    lane = lax.iota(jnp.int32, 16)
    src0 = lane * EMBED_DIM
    src1 = jnp.minimum(lane + 16, NUM_GENRES - 1) * EMBED_DIM
    zero = jnp.zeros((16,), jnp.float32)

    def norm_body(c, carry):
        a0, a1, s0, s1 = carry
        v0 = plsc.load_gather(tab_v, [s0])
        v1 = plsc.load_gather(tab_v, [s1])
        return (a0 + v0 * v0, a1 + v1 * v1, s0 + 1, s1 + 1)

    acc0, acc1, _, _ = lax.fori_loop(
        0, EMBED_DIM, norm_body, (zero, zero, src0, src1), unroll=8)
    inv_v[pl.ds(0, 16)] = jnp.minimum(_fast_rsqrt(acc0), jnp.float32(1e12))
    inv_v[pl.ds(16, 16)] = jnp.minimum(_fast_rsqrt(acc1), jnp.float32(1e12))

    dst_lane = lane * EMBED_DIM
    copies = []
    for g in range(_GROUPS):
        idxv = idx_v[pl.ds(g * 16, 16)]
        scale = plsc.load_gather(inv_v, [idxv])
        src = idxv * EMBED_DIM
        dst = dst_lane + g * (16 * EMBED_DIM)

        def col_body(c, carry, scale=scale):
            s, d = carry
            v = plsc.load_gather(tab_v, [s])
            plsc.store_scatter(rows_v, [d], v * scale)
            return (s + 1, d + 1)

        lax.fori_loop(0, EMBED_DIM, col_body, (src, dst), unroll=12)

        if g % _CHUNK_GROUPS == _CHUNK_GROUPS - 1:
            k = g // _CHUNK_GROUPS
            off = k * _CHUNK_ELEMS
            copies.append(pltpu.async_copy(
                rows_v.at[pl.ds(off, _CHUNK_ELEMS)],
                out_hbm.at[pl.ds(base * EMBED_DIM + off, _CHUNK_ELEMS)],
                sem))
    for c in copies:
        c.wait()


_sc_user_tower = functools.partial(
    pl.kernel,
    out_type=jax.ShapeDtypeStruct((BATCH * EMBED_DIM,), jnp.float32),
    mesh=plsc.VectorSubcoreMesh(core_axis_name="c", subcore_axis_name="s"),
    scratch_types=[
        pltpu.VMEM((_B_PER_W,), jnp.int32),
        pltpu.VMEM((NUM_GENRES * EMBED_DIM,), jnp.float32),
        pltpu.VMEM((32,), jnp.float32),
        pltpu.VMEM((_B_PER_W * EMBED_DIM,), jnp.float32),
        pltpu.SemaphoreType.DMA,
    ],
    compiler_params=pltpu.CompilerParams(
        use_tc_tiling_on_sc=False, needs_layout_passes=False),
)(_sc_user_body)


_BM = 1024


def _movie_body(title_ref, feat_ref, wt_ref, wm_ref, b_ref, out_ref):
    acc = jnp.dot(title_ref[...], wt_ref[...], preferred_element_type=jnp.float32)
    acc = acc + jnp.dot(feat_ref[...], wm_ref[...], preferred_element_type=jnp.float32)
    acc = acc + b_ref[...]
    norm = jnp.sqrt(jnp.sum(acc * acc, axis=1, keepdims=True))
    out_ref[...] = acc / jnp.maximum(norm, 1e-12)


def _movie_tower(title_embeddings, movie_features, W_movie, b_movie):
    w_t = W_movie[:TITLE_DIM]
    w_m = W_movie[TITLE_DIM:]
    bias = b_movie.reshape(1, EMBED_DIM)
    grid = (BATCH // _BM,)
    return pl.pallas_call(
        _movie_body,
        grid=grid,
        in_specs=[
            pl.BlockSpec((_BM, TITLE_DIM), lambda i: (i, 0)),
            pl.BlockSpec((_BM, MOVIE_FEAT_DIM), lambda i: (i, 0)),
            pl.BlockSpec((TITLE_DIM, EMBED_DIM), lambda i: (0, 0)),
            pl.BlockSpec((MOVIE_FEAT_DIM, EMBED_DIM), lambda i: (0, 0)),
            pl.BlockSpec((1, EMBED_DIM), lambda i: (0, 0)),
        ],
        out_specs=pl.BlockSpec((_BM, EMBED_DIM), lambda i: (i, 0)),
        out_shape=jax.ShapeDtypeStruct((BATCH, EMBED_DIM), jnp.float32),
    )(title_embeddings, movie_features, w_t, w_m, bias)


def kernel(user_features, title_embeddings, movie_features, user_table, W_movie, b_movie):
    user_flat = _sc_user_tower(user_features, user_table.reshape(-1))
    user_embedding = user_flat.reshape(BATCH, EMBED_DIM)
    movie_embedding = _movie_tower(title_embeddings, movie_features, W_movie, b_movie)
    return (user_embedding, movie_embedding)


# speedup vs baseline: 1.2759x; 1.2759x over previous
import functools

import jax
import jax.numpy as jnp
from jax import lax
from jax.experimental import pallas as pl
from jax.experimental.pallas import tpu as pltpu
from jax.experimental.pallas import tpu_sc as plsc

NUM_GENRES = 20
EMBED_DIM = 240
TITLE_DIM = 768
MOVIE_FEAT_DIM = 64
BATCH = 16384

_NC = 2
_NS = 16
_NW = _NC * _NS
_B_PER_W = BATCH // _NW
_GROUPS = _B_PER_W // 16
_CHUNK_GROUPS = 8
_N_CHUNKS = _GROUPS // _CHUNK_GROUPS
_CHUNK_ELEMS = _CHUNK_GROUPS * 16 * EMBED_DIM


def _fast_rsqrt(s):
    i = plsc.bitcast(s, jnp.int32)
    i = jnp.int32(0x5F3759DF) - (i >> 1)
    y = plsc.bitcast(i, jnp.float32)
    for _ in range(3):
        y = y * (jnp.float32(1.5) - jnp.float32(0.5) * s * y * y)
    return y


_CHUNK_ROWS = _CHUNK_GROUPS * 16


def _sc_user_body(idx_hbm, tab_hbm, out_hbm, idx_v, tab_v, inv_v, rows_a, rows_b, sem):
    wid = lax.axis_index("s") * _NC + lax.axis_index("c")
    base = wid * _B_PER_W

    pltpu.sync_copy(idx_hbm.at[pl.ds(base, _B_PER_W)], idx_v)
    pltpu.sync_copy(tab_hbm, tab_v)

    lane = lax.iota(jnp.int32, 16)
    rows0 = lane
    rows1 = jnp.minimum(lane + 16, NUM_GENRES - 1)
    zero = jnp.zeros((16,), jnp.float32)

    def norm_body(c, carry):
        a0, a1 = carry
        cc = jnp.full((16,), c, jnp.int32)
        v0 = plsc.load_gather(tab_v, [rows0, cc])
        v1 = plsc.load_gather(tab_v, [rows1, cc])
        return (a0 + v0 * v0, a1 + v1 * v1)

    acc0, acc1 = plsc.parallel_loop(
        0, EMBED_DIM, unroll=8, carry=(zero, zero))(norm_body)
    inv_v[pl.ds(0, 16)] = jnp.minimum(_fast_rsqrt(acc0), jnp.float32(1e12))
    inv_v[pl.ds(16, 16)] = jnp.minimum(_fast_rsqrt(acc1), jnp.float32(1e12))

    bufs = (rows_a, rows_b)
    copies = []
    for g in range(_GROUPS):
        k = g // _CHUNK_GROUPS
        buf = bufs[k % 2]
        if g % _CHUNK_GROUPS == 0 and k >= 2:
            copies[k - 2].wait()
        idxv = idx_v[pl.ds(g * 16, 16)]
        scale = plsc.load_gather(inv_v, [idxv])
        dstrow = (g % _CHUNK_GROUPS) * 16 + lane

        def col_body(c, idxv=idxv, scale=scale, dstrow=dstrow, buf=buf):
            cc = jnp.full((16,), c, jnp.int32)
            v = plsc.load_gather(tab_v, [idxv, cc])
            plsc.store_scatter(buf, [dstrow, cc], v * scale)

        plsc.parallel_loop(0, EMBED_DIM, unroll=8)(col_body)

        if g % _CHUNK_GROUPS == _CHUNK_GROUPS - 1:
            copies.append(pltpu.async_copy(
                buf,
                out_hbm.at[pl.ds(base + k * _CHUNK_ROWS, _CHUNK_ROWS)],
                sem))
    for c in copies[-2:]:
        c.wait()


_sc_user_tower = functools.partial(
    pl.kernel,
    out_type=jax.ShapeDtypeStruct((BATCH, EMBED_DIM), jnp.float32),
    mesh=plsc.VectorSubcoreMesh(core_axis_name="c", subcore_axis_name="s"),
    scratch_types=[
        pltpu.VMEM((_B_PER_W,), jnp.int32),
        pltpu.VMEM((NUM_GENRES, EMBED_DIM), jnp.float32),
        pltpu.VMEM((32,), jnp.float32),
        pltpu.VMEM((_CHUNK_ROWS, EMBED_DIM), jnp.float32),
        pltpu.VMEM((_CHUNK_ROWS, EMBED_DIM), jnp.float32),
        pltpu.SemaphoreType.DMA,
    ],
    compiler_params=pltpu.CompilerParams(needs_layout_passes=False),
)(_sc_user_body)


_BM = 1024


def _movie_body(title_ref, feat_ref, wt_ref, wm_ref, b_ref, out_ref):
    acc = jnp.dot(title_ref[...], wt_ref[...], preferred_element_type=jnp.float32)
    acc = acc + jnp.dot(feat_ref[...], wm_ref[...], preferred_element_type=jnp.float32)
    acc = acc + b_ref[...]
    norm = jnp.sqrt(jnp.sum(acc * acc, axis=1, keepdims=True))
    out_ref[...] = acc / jnp.maximum(norm, 1e-12)


def _movie_tower(title_embeddings, movie_features, W_movie, b_movie):
    w_t = W_movie[:TITLE_DIM]
    w_m = W_movie[TITLE_DIM:]
    bias = b_movie.reshape(1, EMBED_DIM)
    grid = (BATCH // _BM,)
    return pl.pallas_call(
        _movie_body,
        grid=grid,
        in_specs=[
            pl.BlockSpec((_BM, TITLE_DIM), lambda i: (i, 0)),
            pl.BlockSpec((_BM, MOVIE_FEAT_DIM), lambda i: (i, 0)),
            pl.BlockSpec((TITLE_DIM, EMBED_DIM), lambda i: (0, 0)),
            pl.BlockSpec((MOVIE_FEAT_DIM, EMBED_DIM), lambda i: (0, 0)),
            pl.BlockSpec((1, EMBED_DIM), lambda i: (0, 0)),
        ],
        out_specs=pl.BlockSpec((_BM, EMBED_DIM), lambda i: (i, 0)),
        out_shape=jax.ShapeDtypeStruct((BATCH, EMBED_DIM), jnp.float32),
    )(title_embeddings, movie_features, w_t, w_m, bias)


def kernel(user_features, title_embeddings, movie_features, user_table, W_movie, b_movie):
    user_embedding = _sc_user_tower(user_features, user_table)
    movie_embedding = _movie_tower(title_embeddings, movie_features, W_movie, b_movie)
    return (user_embedding, movie_embedding)


# speedup vs baseline: 1.6656x; 1.3054x over previous
import functools

import jax
import jax.numpy as jnp
from jax import lax
from jax.experimental import pallas as pl
from jax.experimental.pallas import tpu as pltpu
from jax.experimental.pallas import tpu_sc as plsc

NUM_GENRES = 20
EMBED_DIM = 240
TITLE_DIM = 768
MOVIE_FEAT_DIM = 64
BATCH = 16384

_NC = 2
_NS = 16
_NW = _NC * _NS
_B_PER_W = BATCH // _NW
_GROUPS = _B_PER_W // 16
_CHUNK_GROUPS = 8
_N_CHUNKS = _GROUPS // _CHUNK_GROUPS
_CHUNK_ELEMS = _CHUNK_GROUPS * 16 * EMBED_DIM


def _fast_rsqrt(s):
    i = plsc.bitcast(s, jnp.int32)
    i = jnp.int32(0x5F3759DF) - (i >> 1)
    y = plsc.bitcast(i, jnp.float32)
    for _ in range(3):
        y = y * (jnp.float32(1.5) - jnp.float32(0.5) * s * y * y)
    return y


_CHUNK_ROWS = _CHUNK_GROUPS * 16


_NCHUNK = EMBED_DIM // 16


def _sc_user_body(idx_hbm, tab_hbm, out_hbm,
                  idx_v, idx_smem, inv_smem, tab_v, rows_a, rows_b, sem):
    wid = lax.axis_index("s") * _NC + lax.axis_index("c")
    base = wid * _B_PER_W

    pltpu.sync_copy(idx_hbm.at[pl.ds(base, _B_PER_W)], idx_v)
    pltpu.sync_copy(tab_hbm, tab_v)

    def inv_body(r, _):
        acc = jnp.zeros((16,), jnp.float32)
        for k in range(_NCHUNK):
            v = tab_v[pl.ds(r * EMBED_DIM + k * 16, 16)]
            acc = acc + v * v
        s = jnp.sum(acc)
        i = lax.bitcast_convert_type(s, jnp.int32)
        i = jnp.int32(0x5F3759DF) - (i >> 1)
        y = lax.bitcast_convert_type(i, jnp.float32)
        for _ in range(3):
            y = y * (jnp.float32(1.5) - jnp.float32(0.5) * s * y * y)
        inv_smem[r] = jnp.minimum(y, jnp.float32(1e12))
        return 0

    lax.fori_loop(0, NUM_GENRES, inv_body, 0)

    def spill_body(g, _):
        idxv = idx_v[pl.ds(g * 16, 16)]
        for l in range(16):
            idx_smem[g * 16 + l] = idxv[l]
        return 0

    lax.fori_loop(0, _GROUPS, spill_body, 0)

    bufs = (rows_a, rows_b)
    copies = []
    for k in range(_N_CHUNKS):
        buf = bufs[k % 2]
        if k >= 2:
            copies[k - 2].wait()

        def row_body(i, k=k, buf=buf):
            r = idx_smem[k * _CHUNK_ROWS + i]
            s = inv_smem[r]
            src = pl.multiple_of(r * EMBED_DIM, 16)
            dst = pl.multiple_of(i * EMBED_DIM, 16)
            for c in range(_NCHUNK):
                buf[pl.ds(dst + c * 16, 16)] = (
                    tab_v[pl.ds(src + c * 16, 16)] * s)

        plsc.parallel_loop(0, _CHUNK_ROWS, unroll=2)(row_body)
        copies.append(pltpu.async_copy(
            buf,
            out_hbm.at[pl.ds((base + k * _CHUNK_ROWS) * EMBED_DIM,
                             _CHUNK_ROWS * EMBED_DIM)],
            sem))
    for c in copies[-2:]:
        c.wait()


_sc_user_tower = functools.partial(
    pl.kernel,
    out_type=jax.ShapeDtypeStruct((BATCH * EMBED_DIM,), jnp.float32),
    mesh=plsc.VectorSubcoreMesh(core_axis_name="c", subcore_axis_name="s"),
    scratch_types=[
        pltpu.VMEM((_B_PER_W,), jnp.int32),
        pltpu.SMEM((_B_PER_W,), jnp.int32),
        pltpu.SMEM((NUM_GENRES,), jnp.float32),
        pltpu.VMEM((NUM_GENRES * EMBED_DIM,), jnp.float32),
        pltpu.VMEM((_CHUNK_ROWS * EMBED_DIM,), jnp.float32),
        pltpu.VMEM((_CHUNK_ROWS * EMBED_DIM,), jnp.float32),
        pltpu.SemaphoreType.DMA,
    ],
    compiler_params=pltpu.CompilerParams(
        use_tc_tiling_on_sc=False, needs_layout_passes=False),
)(_sc_user_body)


_BM = 1024


def _movie_body(title_ref, feat_ref, wt_ref, wm_ref, b_ref, out_ref):
    acc = jnp.dot(title_ref[...], wt_ref[...], preferred_element_type=jnp.float32)
    acc = acc + jnp.dot(feat_ref[...], wm_ref[...], preferred_element_type=jnp.float32)
    acc = acc + b_ref[...]
    norm = jnp.sqrt(jnp.sum(acc * acc, axis=1, keepdims=True))
    out_ref[...] = acc / jnp.maximum(norm, 1e-12)


def _movie_tower(title_embeddings, movie_features, W_movie, b_movie):
    w_t = W_movie[:TITLE_DIM]
    w_m = W_movie[TITLE_DIM:]
    bias = b_movie.reshape(1, EMBED_DIM)
    grid = (BATCH // _BM,)
    return pl.pallas_call(
        _movie_body,
        grid=grid,
        in_specs=[
            pl.BlockSpec((_BM, TITLE_DIM), lambda i: (i, 0)),
            pl.BlockSpec((_BM, MOVIE_FEAT_DIM), lambda i: (i, 0)),
            pl.BlockSpec((TITLE_DIM, EMBED_DIM), lambda i: (0, 0)),
            pl.BlockSpec((MOVIE_FEAT_DIM, EMBED_DIM), lambda i: (0, 0)),
            pl.BlockSpec((1, EMBED_DIM), lambda i: (0, 0)),
        ],
        out_specs=pl.BlockSpec((_BM, EMBED_DIM), lambda i: (i, 0)),
        out_shape=jax.ShapeDtypeStruct((BATCH, EMBED_DIM), jnp.float32),
    )(title_embeddings, movie_features, w_t, w_m, bias)


def kernel(user_features, title_embeddings, movie_features, user_table, W_movie, b_movie):
    user_flat = _sc_user_tower(user_features, user_table.reshape(-1))
    user_embedding = user_flat.reshape(BATCH, EMBED_DIM)
    movie_embedding = _movie_tower(title_embeddings, movie_features, W_movie, b_movie)
    return (user_embedding, movie_embedding)


# speedup vs baseline: 2.0987x; 1.2600x over previous
import functools

import jax
import jax.numpy as jnp
from jax import lax
from jax.experimental import pallas as pl
from jax.experimental.pallas import tpu as pltpu
from jax.experimental.pallas import tpu_sc as plsc

NUM_GENRES = 20
EMBED_DIM = 240
TITLE_DIM = 768
MOVIE_FEAT_DIM = 64
BATCH = 16384

_NC = 2
_NS = 16
_NW = _NC * _NS
_B_PER_W = BATCH // _NW
_GROUPS = _B_PER_W // 16
_CHUNK_GROUPS = 8
_N_CHUNKS = _GROUPS // _CHUNK_GROUPS
_CHUNK_ELEMS = _CHUNK_GROUPS * 16 * EMBED_DIM


def _fast_rsqrt(s):
    i = plsc.bitcast(s, jnp.int32)
    i = jnp.int32(0x5F3759DF) - (i >> 1)
    y = plsc.bitcast(i, jnp.float32)
    for _ in range(3):
        y = y * (jnp.float32(1.5) - jnp.float32(0.5) * s * y * y)
    return y


_CHUNK_ROWS = _CHUNK_GROUPS * 16


_NCHUNK = EMBED_DIM // 16


def _sc_user_body(idx_hbm, tab_hbm, out_hbm,
                  idx_v, idx_smem, inv_smem, tab_v, rows_a, rows_b, sem):
    wid = lax.axis_index("s") * _NC + lax.axis_index("c")
    base = wid * _B_PER_W

    pltpu.sync_copy(idx_hbm.at[pl.ds(base, _B_PER_W)], idx_v)
    pltpu.sync_copy(tab_hbm, tab_v)

    def inv_body(r, _):
        acc = jnp.zeros((16,), jnp.float32)
        for k in range(_NCHUNK):
            v = tab_v[r, pl.ds(k * 16, 16)]
            acc = acc + v * v
        s = jnp.sum(acc)
        i = lax.bitcast_convert_type(s, jnp.int32)
        i = jnp.int32(0x5F3759DF) - (i >> 1)
        y = lax.bitcast_convert_type(i, jnp.float32)
        for _ in range(3):
            y = y * (jnp.float32(1.5) - jnp.float32(0.5) * s * y * y)
        inv_smem[r] = jnp.minimum(y, jnp.float32(1e12))
        return 0

    lax.fori_loop(0, NUM_GENRES, inv_body, 0)

    def spill_body(g, _):
        idxv = idx_v[pl.ds(g * 16, 16)]
        for l in range(16):
            idx_smem[g * 16 + l] = idxv[l]
        return 0

    lax.fori_loop(0, _GROUPS, spill_body, 0)

    bufs = (rows_a, rows_b)
    copies = []
    for k in range(_N_CHUNKS):
        buf = bufs[k % 2]
        if k >= 2:
            copies[k - 2].wait()

        def row_body(i, k=k, buf=buf):
            r = idx_smem[k * _CHUNK_ROWS + i]
            s = inv_smem[r]
            for c in range(_NCHUNK):
                buf[i, pl.ds(c * 16, 16)] = tab_v[r, pl.ds(c * 16, 16)] * s

        plsc.parallel_loop(0, _CHUNK_ROWS, unroll=2)(row_body)
        copies.append(pltpu.async_copy(
            buf,
            out_hbm.at[pl.ds(base + k * _CHUNK_ROWS, _CHUNK_ROWS)],
            sem))
    for c in copies[-2:]:
        c.wait()


_sc_user_tower = functools.partial(
    pl.kernel,
    out_type=jax.ShapeDtypeStruct((BATCH, EMBED_DIM), jnp.float32),
    mesh=plsc.VectorSubcoreMesh(core_axis_name="c", subcore_axis_name="s"),
    scratch_types=[
        pltpu.VMEM((_B_PER_W,), jnp.int32),
        pltpu.SMEM((_B_PER_W,), jnp.int32),
        pltpu.SMEM((NUM_GENRES,), jnp.float32),
        pltpu.VMEM((NUM_GENRES, EMBED_DIM), jnp.float32),
        pltpu.VMEM((_CHUNK_ROWS, EMBED_DIM), jnp.float32),
        pltpu.VMEM((_CHUNK_ROWS, EMBED_DIM), jnp.float32),
        pltpu.SemaphoreType.DMA,
    ],
    compiler_params=pltpu.CompilerParams(needs_layout_passes=False),
)(_sc_user_body)


_BM = 1024


def _movie_body(title_ref, feat_ref, wt_ref, wm_ref, b_ref, out_ref):
    acc = jnp.dot(title_ref[...], wt_ref[...], preferred_element_type=jnp.float32)
    acc = acc + jnp.dot(feat_ref[...], wm_ref[...], preferred_element_type=jnp.float32)
    acc = acc + b_ref[...]
    norm = jnp.sqrt(jnp.sum(acc * acc, axis=1, keepdims=True))
    out_ref[...] = acc / jnp.maximum(norm, 1e-12)


def _movie_tower(title_embeddings, movie_features, W_movie, b_movie):
    w_t = W_movie[:TITLE_DIM]
    w_m = W_movie[TITLE_DIM:]
    bias = b_movie.reshape(1, EMBED_DIM)
    grid = (BATCH // _BM,)
    return pl.pallas_call(
        _movie_body,
        grid=grid,
        in_specs=[
            pl.BlockSpec((_BM, TITLE_DIM), lambda i: (i, 0)),
            pl.BlockSpec((_BM, MOVIE_FEAT_DIM), lambda i: (i, 0)),
            pl.BlockSpec((TITLE_DIM, EMBED_DIM), lambda i: (0, 0)),
            pl.BlockSpec((MOVIE_FEAT_DIM, EMBED_DIM), lambda i: (0, 0)),
            pl.BlockSpec((1, EMBED_DIM), lambda i: (0, 0)),
        ],
        out_specs=pl.BlockSpec((_BM, EMBED_DIM), lambda i: (i, 0)),
        out_shape=jax.ShapeDtypeStruct((BATCH, EMBED_DIM), jnp.float32),
    )(title_embeddings, movie_features, w_t, w_m, bias)


def kernel(user_features, title_embeddings, movie_features, user_table, W_movie, b_movie):
    user_embedding = _sc_user_tower(user_features, user_table)
    movie_embedding = _movie_tower(title_embeddings, movie_features, W_movie, b_movie)
    return (user_embedding, movie_embedding)


# speedup vs baseline: 2.7034x; 1.2882x over previous
import functools

import jax
import jax.numpy as jnp
from jax import lax
from jax.experimental import pallas as pl
from jax.experimental.pallas import tpu as pltpu
from jax.experimental.pallas import tpu_sc as plsc

NUM_GENRES = 20
EMBED_DIM = 240
TITLE_DIM = 768
MOVIE_FEAT_DIM = 64
BATCH = 16384

_NC = 2
_NS = 16
_NW = _NC * _NS
_B_PER_W = BATCH // _NW
_GROUPS = _B_PER_W // 16
_CHUNK_GROUPS = 8
_N_CHUNKS = _GROUPS // _CHUNK_GROUPS
_CHUNK_ELEMS = _CHUNK_GROUPS * 16 * EMBED_DIM


def _fast_rsqrt(s):
    i = plsc.bitcast(s, jnp.int32)
    i = jnp.int32(0x5F3759DF) - (i >> 1)
    y = plsc.bitcast(i, jnp.float32)
    for _ in range(3):
        y = y * (jnp.float32(1.5) - jnp.float32(0.5) * s * y * y)
    return y


_CHUNK_ROWS = _CHUNK_GROUPS * 16


_NCHUNK = EMBED_DIM // 16


def _sc_user_body(idx_hbm, tab_hbm, out_hbm,
                  idx_v, idx_smem, inv_smem, tab_v, rows_a, rows_b, sem):
    wid = lax.axis_index("s") * _NC + lax.axis_index("c")
    base = wid * _B_PER_W

    pltpu.sync_copy(idx_hbm.at[pl.ds(base, _B_PER_W)], idx_v)
    pltpu.sync_copy(tab_hbm, tab_v)

    def inv_body(r, _):
        acc = jnp.zeros((16,), jnp.float32)
        for k in range(_NCHUNK):
            v = tab_v[r, pl.ds(k * 16, 16)]
            acc = acc + v * v
        s = jnp.sum(acc)
        i = lax.bitcast_convert_type(s, jnp.int32)
        i = jnp.int32(0x5F3759DF) - (i >> 1)
        y = lax.bitcast_convert_type(i, jnp.float32)
        for _ in range(3):
            y = y * (jnp.float32(1.5) - jnp.float32(0.5) * s * y * y)
        inv_smem[r] = jnp.minimum(y, jnp.float32(1e12))
        return 0

    lax.fori_loop(0, NUM_GENRES, inv_body, 0)

    def spill_body(g, _):
        idxv = idx_v[pl.ds(g * 16, 16)]
        for l in range(16):
            idx_smem[g * 16 + l] = idxv[l]
        return 0

    lax.fori_loop(0, _GROUPS, spill_body, 0)

    bufs = (rows_a, rows_b)
    copies = []
    for k in range(_N_CHUNKS):
        buf = bufs[k % 2]
        if k >= 2:
            copies[k - 2].wait()

        def row_body(i, k=k, buf=buf):
            r = idx_smem[k * _CHUNK_ROWS + i]
            s = inv_smem[r]
            for c in range(_NCHUNK):
                buf[i, pl.ds(c * 16, 16)] = tab_v[r, pl.ds(c * 16, 16)] * s

        plsc.parallel_loop(0, _CHUNK_ROWS, unroll=2)(row_body)
        copies.append(pltpu.async_copy(
            buf,
            out_hbm.at[pl.ds(base + k * _CHUNK_ROWS, _CHUNK_ROWS)],
            sem))
    for c in copies[-2:]:
        c.wait()


_sc_user_tower = functools.partial(
    pl.kernel,
    out_type=jax.ShapeDtypeStruct((BATCH, EMBED_DIM), jnp.float32),
    mesh=plsc.VectorSubcoreMesh(core_axis_name="c", subcore_axis_name="s"),
    scratch_types=[
        pltpu.VMEM((_B_PER_W,), jnp.int32),
        pltpu.SMEM((_B_PER_W,), jnp.int32),
        pltpu.SMEM((NUM_GENRES,), jnp.float32),
        pltpu.VMEM((NUM_GENRES, EMBED_DIM), jnp.float32),
        pltpu.VMEM((_CHUNK_ROWS, EMBED_DIM), jnp.float32),
        pltpu.VMEM((_CHUNK_ROWS, EMBED_DIM), jnp.float32),
        pltpu.SemaphoreType.DMA,
    ],
    compiler_params=pltpu.CompilerParams(needs_layout_passes=False),
)(_sc_user_body)


_BM = 1024


def _movie_body(title_ref, feat_t_ref, wt_t_ref, wm_t_ref, b_ref, out_ref):
    acc = lax.dot_general(
        wt_t_ref[...], title_ref[...],
        dimension_numbers=(((1,), (1,)), ((), ())),
        preferred_element_type=jnp.float32)
    acc = acc + jnp.dot(wm_t_ref[...], feat_t_ref[...],
                        preferred_element_type=jnp.float32)
    acc = acc + b_ref[...]
    norm = jnp.sqrt(jnp.sum(acc * acc, axis=0, keepdims=True))
    out_ref[...] = acc / jnp.maximum(norm, 1e-12)


def _movie_tower(title_embeddings, movie_features, W_movie, b_movie):
    w_T = W_movie.T
    wt_t = w_T[:, :TITLE_DIM]
    wm_t = w_T[:, TITLE_DIM:]
    feat_t = movie_features.T
    bias = b_movie.reshape(EMBED_DIM, 1)
    grid = (BATCH // _BM,)
    out_t = pl.pallas_call(
        _movie_body,
        grid=grid,
        in_specs=[
            pl.BlockSpec((_BM, TITLE_DIM), lambda i: (i, 0)),
            pl.BlockSpec((MOVIE_FEAT_DIM, _BM), lambda i: (0, i)),
            pl.BlockSpec((EMBED_DIM, TITLE_DIM), lambda i: (0, 0)),
            pl.BlockSpec((EMBED_DIM, MOVIE_FEAT_DIM), lambda i: (0, 0)),
            pl.BlockSpec((EMBED_DIM, 1), lambda i: (0, 0)),
        ],
        out_specs=pl.BlockSpec((EMBED_DIM, _BM), lambda i: (0, i)),
        out_shape=jax.ShapeDtypeStruct((EMBED_DIM, BATCH), jnp.float32),
    )(title_embeddings, feat_t, wt_t, wm_t, bias)
    return out_t.T


def kernel(user_features, title_embeddings, movie_features, user_table, W_movie, b_movie):
    user_embedding = _sc_user_tower(user_features, user_table)
    movie_embedding = _movie_tower(title_embeddings, movie_features, W_movie, b_movie)
    return (user_embedding, movie_embedding)


# speedup vs baseline: 3.4956x; 1.2930x over previous
import functools

import jax
import jax.numpy as jnp
from jax import lax
from jax.experimental import pallas as pl
from jax.experimental.pallas import tpu as pltpu
from jax.experimental.pallas import tpu_sc as plsc

NUM_GENRES = 20
EMBED_DIM = 240
TITLE_DIM = 768
MOVIE_FEAT_DIM = 64
BATCH = 16384

_NC = 2
_NS = 16
_NW = _NC * _NS
_B_PER_W = BATCH // _NW
_GROUPS = _B_PER_W // 16
_CHUNK_GROUPS = 8
_N_CHUNKS = _GROUPS // _CHUNK_GROUPS
_CHUNK_ELEMS = _CHUNK_GROUPS * 16 * EMBED_DIM


_CHUNK_ROWS = _CHUNK_GROUPS * 16
_NCHUNK = EMBED_DIM // 16
_PAD_STRIDE = EMBED_DIM + 1


def _fast_rsqrt(s):
    i = plsc.bitcast(s, jnp.int32)
    i = jnp.int32(0x5F3759DF) - (i >> 1)
    y = plsc.bitcast(i, jnp.float32)
    for _ in range(3):
        y = y * (jnp.float32(1.5) - jnp.float32(0.5) * s * y * y)
    return y


def _sc_user_body(idx_hbm, tab_hbm, out_hbm,
                  idx_v, tab2d, tab_flat, inv_v, buf_a, buf_b, sem):
    wid = lax.axis_index("s") * _NC + lax.axis_index("c")
    base = wid * _B_PER_W
    lane = lax.iota(jnp.int32, 16)

    pltpu.sync_copy(idx_hbm.at[pl.ds(base, _B_PER_W)], idx_v)
    pltpu.sync_copy(tab_hbm, tab2d)

    def repack_body(r, _):
        for c in range(_NCHUNK):
            v = tab2d[r, pl.ds(c * 16, 16)]
            plsc.store_scatter(
                tab_flat, [r * _PAD_STRIDE + c * 16 + lane], v)
        return 0

    lax.fori_loop(0, NUM_GENRES, repack_body, 0)

    src0 = lane * _PAD_STRIDE
    src1 = jnp.minimum(lane + 16, NUM_GENRES - 1) * _PAD_STRIDE
    zero = jnp.zeros((16,), jnp.float32)

    def norm_body(c, carry):
        a0, a1 = carry
        v0 = plsc.load_gather(tab_flat, [src0 + c])
        v1 = plsc.load_gather(tab_flat, [src1 + c])
        return (a0 + v0 * v0, a1 + v1 * v1)

    acc0, acc1 = plsc.parallel_loop(
        0, EMBED_DIM, unroll=8, carry=(zero, zero))(norm_body)
    inv_v[pl.ds(0, 16)] = jnp.minimum(_fast_rsqrt(acc0), jnp.float32(1e12))
    inv_v[pl.ds(16, 16)] = jnp.minimum(_fast_rsqrt(acc1), jnp.float32(1e12))

    bufs = (buf_a, buf_b)
    copies = []
    for k in range(_N_CHUNKS):
        buf = bufs[k % 2]
        if k >= 2:
            copies[k - 2].wait()
        for j in range(_CHUNK_GROUPS):
            g = k * _CHUNK_GROUPS + j
            idxv = idx_v[pl.ds(g * 16, 16)]
            scale = plsc.load_gather(inv_v, [idxv])
            src = idxv * _PAD_STRIDE

            def col_body(d, j=j, src=src, scale=scale, buf=buf):
                v = plsc.load_gather(tab_flat, [src + d])
                buf[d, pl.ds(j * 16, 16)] = v * scale

            plsc.parallel_loop(0, EMBED_DIM, unroll=8)(col_body)
        copies.append(pltpu.async_copy(
            buf,
            out_hbm.at[:, pl.ds(base + k * _CHUNK_ROWS, _CHUNK_ROWS)],
            sem))
    for c in copies[-2:]:
        c.wait()


_sc_user_tower = functools.partial(
    pl.kernel,
    out_type=jax.ShapeDtypeStruct((EMBED_DIM, BATCH), jnp.float32),
    mesh=plsc.VectorSubcoreMesh(core_axis_name="c", subcore_axis_name="s"),
    scratch_types=[
        pltpu.VMEM((_B_PER_W,), jnp.int32),
        pltpu.VMEM((NUM_GENRES, EMBED_DIM), jnp.float32),
        pltpu.VMEM((NUM_GENRES * _PAD_STRIDE + 12,), jnp.float32),
        pltpu.VMEM((32,), jnp.float32),
        pltpu.VMEM((EMBED_DIM, _CHUNK_ROWS), jnp.float32),
        pltpu.VMEM((EMBED_DIM, _CHUNK_ROWS), jnp.float32),
        pltpu.SemaphoreType.DMA,
    ],
    compiler_params=pltpu.CompilerParams(needs_layout_passes=False),
)(_sc_user_body)


_BM = 1024


def _movie_body(title_ref, feat_t_ref, wt_t_ref, wm_t_ref, b_ref, out_ref):
    acc = lax.dot_general(
        wt_t_ref[...], title_ref[...],
        dimension_numbers=(((1,), (1,)), ((), ())),
        preferred_element_type=jnp.float32)
    acc = acc + jnp.dot(wm_t_ref[...], feat_t_ref[...],
                        preferred_element_type=jnp.float32)
    acc = acc + b_ref[...]
    norm = jnp.sqrt(jnp.sum(acc * acc, axis=0, keepdims=True))
    out_ref[...] = acc / jnp.maximum(norm, 1e-12)


def _movie_tower(title_embeddings, movie_features, W_movie, b_movie):
    w_T = W_movie.T
    wt_t = w_T[:, :TITLE_DIM]
    wm_t = w_T[:, TITLE_DIM:]
    feat_t = movie_features.T
    bias = b_movie.reshape(EMBED_DIM, 1)
    grid = (BATCH // _BM,)
    out_t = pl.pallas_call(
        _movie_body,
        grid=grid,
        in_specs=[
            pl.BlockSpec((_BM, TITLE_DIM), lambda i: (i, 0)),
            pl.BlockSpec((MOVIE_FEAT_DIM, _BM), lambda i: (0, i)),
            pl.BlockSpec((EMBED_DIM, TITLE_DIM), lambda i: (0, 0)),
            pl.BlockSpec((EMBED_DIM, MOVIE_FEAT_DIM), lambda i: (0, 0)),
            pl.BlockSpec((EMBED_DIM, 1), lambda i: (0, 0)),
        ],
        out_specs=pl.BlockSpec((EMBED_DIM, _BM), lambda i: (0, i)),
        out_shape=jax.ShapeDtypeStruct((EMBED_DIM, BATCH), jnp.float32),
    )(title_embeddings, feat_t, wt_t, wm_t, bias)
    return out_t.T


def kernel(user_features, title_embeddings, movie_features, user_table, W_movie, b_movie):
    user_embedding = _sc_user_tower(user_features, user_table).T
    movie_embedding = _movie_tower(title_embeddings, movie_features, W_movie, b_movie)
    return (user_embedding, movie_embedding)


# speedup vs baseline: 3.8320x; 1.0962x over previous
import functools

import jax
import jax.numpy as jnp
from jax import lax
from jax.experimental import pallas as pl
from jax.experimental.pallas import tpu as pltpu
from jax.experimental.pallas import tpu_sc as plsc

NUM_GENRES = 20
EMBED_DIM = 240
TITLE_DIM = 768
MOVIE_FEAT_DIM = 64
BATCH = 16384

_NC = 2
_NS = 16
_NW = _NC * _NS
_B_PER_W = BATCH // _NW
_GROUPS = _B_PER_W // 16
_CHUNK_GROUPS = 8
_N_CHUNKS = _GROUPS // _CHUNK_GROUPS
_CHUNK_ELEMS = _CHUNK_GROUPS * 16 * EMBED_DIM


_CHUNK_ROWS = _CHUNK_GROUPS * 16
_NCHUNK = EMBED_DIM // 16
_PAD_STRIDE = EMBED_DIM + 1


def _fast_rsqrt(s):
    i = plsc.bitcast(s, jnp.int32)
    i = jnp.int32(0x5F3759DF) - (i >> 1)
    y = plsc.bitcast(i, jnp.float32)
    for _ in range(3):
        y = y * (jnp.float32(1.5) - jnp.float32(0.5) * s * y * y)
    return y


def _sc_user_body(idx_hbm, tab_hbm, out_hbm,
                  idx_v, tab2d, tab_flat, inv_v, buf_a, buf_b, sem):
    wid = lax.axis_index("s") * _NC + lax.axis_index("c")
    base = wid * _B_PER_W
    lane = lax.iota(jnp.int32, 16)

    pltpu.sync_copy(idx_hbm.at[pl.ds(base, _B_PER_W)], idx_v)
    pltpu.sync_copy(tab_hbm, tab2d)

    def repack_body(r, _):
        for c in range(_NCHUNK):
            v = tab2d[r, pl.ds(c * 16, 16)]
            plsc.store_scatter(
                tab_flat, [r * _PAD_STRIDE + c * 16 + lane], v)
        return 0

    lax.fori_loop(0, NUM_GENRES, repack_body, 0)

    src0 = lane * _PAD_STRIDE
    src1 = jnp.minimum(lane + 16, NUM_GENRES - 1) * _PAD_STRIDE
    zero = jnp.zeros((16,), jnp.float32)

    def norm_body(c, carry):
        a0, a1 = carry
        v0 = plsc.load_gather(tab_flat, [src0 + c])
        v1 = plsc.load_gather(tab_flat, [src1 + c])
        return (a0 + v0 * v0, a1 + v1 * v1)

    acc0, acc1 = plsc.parallel_loop(
        0, EMBED_DIM, unroll=8, carry=(zero, zero))(norm_body)
    inv_v[pl.ds(0, 16)] = jnp.minimum(_fast_rsqrt(acc0), jnp.float32(1e12))
    inv_v[pl.ds(16, 16)] = jnp.minimum(_fast_rsqrt(acc1), jnp.float32(1e12))

    bufs = (buf_a, buf_b)
    copies = []
    for k in range(_N_CHUNKS):
        buf = bufs[k % 2]
        if k >= 2:
            copies[k - 2].wait()
        for j in range(_CHUNK_GROUPS):
            g = k * _CHUNK_GROUPS + j
            idxv = idx_v[pl.ds(g * 16, 16)]
            scale = plsc.load_gather(inv_v, [idxv])
            src = idxv * _PAD_STRIDE

            def col_body(d, j=j, src=src, scale=scale, buf=buf):
                v = plsc.load_gather(tab_flat, [src + d])
                buf[d, pl.ds(j * 16, 16)] = v * scale

            plsc.parallel_loop(0, EMBED_DIM, unroll=8)(col_body)
        copies.append(pltpu.async_copy(
            buf,
            out_hbm.at[:, pl.ds(base + k * _CHUNK_ROWS, _CHUNK_ROWS)],
            sem))
    for c in copies[-2:]:
        c.wait()


_sc_user_tower = functools.partial(
    pl.kernel,
    out_type=jax.ShapeDtypeStruct((EMBED_DIM, BATCH), jnp.float32),
    mesh=plsc.VectorSubcoreMesh(core_axis_name="c", subcore_axis_name="s"),
    scratch_types=[
        pltpu.VMEM((_B_PER_W,), jnp.int32),
        pltpu.VMEM((NUM_GENRES, EMBED_DIM), jnp.float32),
        pltpu.VMEM((NUM_GENRES * _PAD_STRIDE + 12,), jnp.float32),
        pltpu.VMEM((32,), jnp.float32),
        pltpu.VMEM((EMBED_DIM, _CHUNK_ROWS), jnp.float32),
        pltpu.VMEM((EMBED_DIM, _CHUNK_ROWS), jnp.float32),
        pltpu.SemaphoreType.DMA,
    ],
    compiler_params=pltpu.CompilerParams(needs_layout_passes=False),
)(_sc_user_body)


_BM = 2048


def _movie_body(title_ref, feat_t_ref, w_ref, b_ref, out_ref):
    w = w_ref[...]
    acc = lax.dot_general(
        w[:TITLE_DIM], title_ref[...],
        dimension_numbers=(((0,), (1,)), ((), ())),
        preferred_element_type=jnp.float32)
    acc = acc + lax.dot_general(
        w[TITLE_DIM:], feat_t_ref[...],
        dimension_numbers=(((0,), (0,)), ((), ())),
        preferred_element_type=jnp.float32)
    acc = acc + b_ref[...]
    norm = jnp.sqrt(jnp.sum(acc * acc, axis=0, keepdims=True))
    out_ref[...] = acc / jnp.maximum(norm, 1e-12)


def _movie_tower(title_embeddings, movie_features, W_movie, b_movie):
    feat_t = movie_features.T
    bias = b_movie.reshape(EMBED_DIM, 1)
    grid = (BATCH // _BM,)
    out_t = pl.pallas_call(
        _movie_body,
        grid=grid,
        in_specs=[
            pl.BlockSpec((_BM, TITLE_DIM), lambda i: (i, 0)),
            pl.BlockSpec((MOVIE_FEAT_DIM, _BM), lambda i: (0, i)),
            pl.BlockSpec((TITLE_DIM + MOVIE_FEAT_DIM, EMBED_DIM),
                         lambda i: (0, 0)),
            pl.BlockSpec((EMBED_DIM, 1), lambda i: (0, 0)),
        ],
        out_specs=pl.BlockSpec((EMBED_DIM, _BM), lambda i: (0, i)),
        out_shape=jax.ShapeDtypeStruct((EMBED_DIM, BATCH), jnp.float32),
    )(title_embeddings, feat_t, W_movie, bias)
    return out_t.T


def kernel(user_features, title_embeddings, movie_features, user_table, W_movie, b_movie):
    user_embedding = _sc_user_tower(user_features, user_table).T
    movie_embedding = _movie_tower(title_embeddings, movie_features, W_movie, b_movie)
    return (user_embedding, movie_embedding)


# speedup vs baseline: 3.8555x; 1.0062x over previous
import functools

import jax
import jax.numpy as jnp
from jax import lax
from jax.experimental import pallas as pl
from jax.experimental.pallas import tpu as pltpu
from jax.experimental.pallas import tpu_sc as plsc

NUM_GENRES = 20
EMBED_DIM = 240
TITLE_DIM = 768
MOVIE_FEAT_DIM = 64
BATCH = 16384

_NC = 2
_NS = 16
_NW = _NC * _NS
_B_PER_W = BATCH // _NW
_GROUPS = _B_PER_W // 16
_CHUNK_GROUPS = 8
_N_CHUNKS = _GROUPS // _CHUNK_GROUPS
_CHUNK_ELEMS = _CHUNK_GROUPS * 16 * EMBED_DIM


_CHUNK_ROWS = _CHUNK_GROUPS * 16
_NCHUNK = EMBED_DIM // 16
_PAD_STRIDE = EMBED_DIM + 1


def _fast_rsqrt(s):
    i = plsc.bitcast(s, jnp.int32)
    i = jnp.int32(0x5F3759DF) - (i >> 1)
    y = plsc.bitcast(i, jnp.float32)
    for _ in range(3):
        y = y * (jnp.float32(1.5) - jnp.float32(0.5) * s * y * y)
    return y


def _sc_user_body(idx_hbm, tab_hbm, out_hbm,
                  idx_v, tab2d, tab_flat, inv_v, buf_a, buf_b, sem):
    wid = lax.axis_index("s") * _NC + lax.axis_index("c")
    base = wid * _B_PER_W
    lane = lax.iota(jnp.int32, 16)

    pltpu.sync_copy(idx_hbm.at[pl.ds(base, _B_PER_W)], idx_v)
    pltpu.sync_copy(tab_hbm, tab2d)

    def repack_body(r, _):
        for c in range(_NCHUNK):
            v = tab2d[r, pl.ds(c * 16, 16)]
            plsc.store_scatter(
                tab_flat, [r * _PAD_STRIDE + c * 16 + lane], v)
        return 0

    lax.fori_loop(0, NUM_GENRES, repack_body, 0)

    src0 = lane * _PAD_STRIDE
    src1 = jnp.minimum(lane + 16, NUM_GENRES - 1) * _PAD_STRIDE
    zero = jnp.zeros((16,), jnp.float32)

    def norm_body(c, carry):
        a0, a1 = carry
        v0 = plsc.load_gather(tab_flat, [src0 + c])
        v1 = plsc.load_gather(tab_flat, [src1 + c])
        return (a0 + v0 * v0, a1 + v1 * v1)

    acc0, acc1 = plsc.parallel_loop(
        0, EMBED_DIM, unroll=8, carry=(zero, zero))(norm_body)
    inv_v[pl.ds(0, 16)] = jnp.minimum(_fast_rsqrt(acc0), jnp.float32(1e12))
    inv_v[pl.ds(16, 16)] = jnp.minimum(_fast_rsqrt(acc1), jnp.float32(1e12))

    bufs = (buf_a, buf_b)
    copies = []
    for k in range(_N_CHUNKS):
        buf = bufs[k % 2]
        if k >= 2:
            copies[k - 2].wait()
        for j in range(_CHUNK_GROUPS):
            g = k * _CHUNK_GROUPS + j
            idxv = idx_v[pl.ds(g * 16, 16)]
            scale = plsc.load_gather(inv_v, [idxv])
            src = idxv * _PAD_STRIDE

            def col_body(d, j=j, src=src, scale=scale, buf=buf):
                v = plsc.load_gather(tab_flat, [src + d])
                buf[d, pl.ds(j * 16, 16)] = v * scale

            plsc.parallel_loop(0, EMBED_DIM, unroll=8)(col_body)
        copies.append(pltpu.async_copy(
            buf,
            out_hbm.at[:, pl.ds(base + k * _CHUNK_ROWS, _CHUNK_ROWS)],
            sem))
    for c in copies[-2:]:
        c.wait()


_sc_user_tower = functools.partial(
    pl.kernel,
    out_type=jax.ShapeDtypeStruct((EMBED_DIM, BATCH), jnp.float32),
    mesh=plsc.VectorSubcoreMesh(core_axis_name="c", subcore_axis_name="s"),
    scratch_types=[
        pltpu.VMEM((_B_PER_W,), jnp.int32),
        pltpu.VMEM((NUM_GENRES, EMBED_DIM), jnp.float32),
        pltpu.VMEM((NUM_GENRES * _PAD_STRIDE + 12,), jnp.float32),
        pltpu.VMEM((32,), jnp.float32),
        pltpu.VMEM((EMBED_DIM, _CHUNK_ROWS), jnp.float32),
        pltpu.VMEM((EMBED_DIM, _CHUNK_ROWS), jnp.float32),
        pltpu.SemaphoreType.DMA,
    ],
    compiler_params=pltpu.CompilerParams(needs_layout_passes=False),
)(_sc_user_body)


_BM = 4096


def _movie_body(title_ref, feat_t_ref, w_ref, b_ref, out_ref):
    w = w_ref[...]
    acc = lax.dot_general(
        w[:TITLE_DIM], title_ref[...],
        dimension_numbers=(((0,), (1,)), ((), ())),
        preferred_element_type=jnp.float32)
    acc = acc + lax.dot_general(
        w[TITLE_DIM:], feat_t_ref[...],
        dimension_numbers=(((0,), (0,)), ((), ())),
        preferred_element_type=jnp.float32)
    acc = acc + b_ref[...]
    norm = jnp.sqrt(jnp.sum(acc * acc, axis=0, keepdims=True))
    out_ref[...] = acc / jnp.maximum(norm, 1e-12)


def _movie_tower(title_embeddings, movie_features, W_movie, b_movie):
    feat_t = movie_features.T
    bias = b_movie.reshape(EMBED_DIM, 1)
    grid = (BATCH // _BM,)
    out_t = pl.pallas_call(
        _movie_body,
        grid=grid,
        in_specs=[
            pl.BlockSpec((_BM, TITLE_DIM), lambda i: (i, 0)),
            pl.BlockSpec((MOVIE_FEAT_DIM, _BM), lambda i: (0, i)),
            pl.BlockSpec((TITLE_DIM + MOVIE_FEAT_DIM, EMBED_DIM),
                         lambda i: (0, 0)),
            pl.BlockSpec((EMBED_DIM, 1), lambda i: (0, 0)),
        ],
        out_specs=pl.BlockSpec((EMBED_DIM, _BM), lambda i: (0, i)),
        out_shape=jax.ShapeDtypeStruct((EMBED_DIM, BATCH), jnp.float32),
    )(title_embeddings, feat_t, W_movie, bias)
    return out_t.T


def kernel(user_features, title_embeddings, movie_features, user_table, W_movie, b_movie):
    user_embedding = _sc_user_tower(user_features, user_table).T
    movie_embedding = _movie_tower(title_embeddings, movie_features, W_movie, b_movie)
    return (user_embedding, movie_embedding)
